# baseline (speedup 1.0000x reference)
import functools

import jax
import jax.numpy as jnp
from jax import lax
from jax.experimental import pallas as pl
from jax.experimental.pallas import tpu as pltpu
from jax.experimental.pallas import tpu_sc as plsc

N_NODES = 10000
N_EDGES = 320000
D = 128

NC = 2
NS = 16
NW = NC * NS
BATCH = 80
CHUNK = 16
N_CHUNK = 8
N_BATCH = CHUNK * N_CHUNK
E_PER_W = BATCH * N_BATCH
E_PAD = NW * E_PER_W
N_PAD = 10240
ROWS_PER_TILE = N_PAD // NS
NBUF = 4


def _sc_aggregate(x, e5, zeros_pad):
    mesh = plsc.VectorSubcoreMesh(core_axis_name="c", subcore_axis_name="s")

    @functools.partial(
        pl.kernel,
        mesh=mesh,
        out_type=jax.ShapeDtypeStruct((NC, N_PAD, D), jnp.float32),
        scratch_types=(
            [pltpu.VMEM((2, CHUNK, 2, BATCH), jnp.int32)]
            + [pltpu.VMEM((BATCH, D), jnp.float32)] * NBUF
            + [pltpu.VMEM_SHARED((N_PAD, D), jnp.float32)]
            + [pltpu.SemaphoreType.DMA] * (2 * NBUF + 2)
        ),
    )
    def agg_kernel(x_hbm, e_hbm, zero_hbm, out_hbm,
                   idx2, r0, r1, r2, r3, acc_sh,
                   g0, g1, g2, g3, s0, s1, s2, s3, i0, i1):
        c = lax.axis_index("c")
        s = lax.axis_index("s")
        wid = c * NS + s
        rows = [r0, r1, r2, r3]
        gsem = [g0, g1, g2, g3]
        ssem = [s0, s1, s2, s3]
        isem = [i0, i1]

        def gather(p, k, b):
            pltpu.async_copy(x_hbm.at[idx2.at[p, k, 0]], rows[b], gsem[b])

        def wait_gather(b):
            pltpu.make_async_copy(x_hbm.at[idx2.at[0, 0, 0]], rows[b],
                                  gsem[b]).wait()

        def scatter(p, k, b):
            pltpu.async_copy(rows[b], acc_sh.at[idx2.at[p, k, 1]], ssem[b],
                             add=True)

        def wait_scatter(b):
            pltpu.make_async_copy(rows[b], acc_sh.at[idx2.at[0, 0, 1]],
                                  ssem[b]).wait()

        def load_chunk(ci, p, sem):
            pltpu.async_copy(e_hbm.at[wid, ci], idx2.at[p], sem)

        def wait_chunk(p):
            pltpu.make_async_copy(e_hbm.at[wid, 0], idx2.at[p],
                                  isem[p]).wait()

        row0 = s * ROWS_PER_TILE
        zcp = pltpu.async_copy(zero_hbm.at[pl.ds(row0, ROWS_PER_TILE)],
                               acc_sh.at[pl.ds(row0, ROWS_PER_TILE)], g0)
        pltpu.sync_copy(e_hbm.at[wid, 0], idx2.at[0])
        zcp.wait()

        for k in range(NBUF):
            gather(0, k, k)
        plsc.subcore_barrier()
        for k in range(2):
            wait_gather(k)
            scatter(0, k, k)
        for k in range(NBUF, CHUNK):
            b = k % NBUF
            wait_scatter(b)
            if k == NBUF:
                load_chunk(1, 1, isem[1])
            gather(0, k, b)
            b2 = (b + 2) % NBUF
            wait_gather(b2)
            scatter(0, k - 2, b2)

        def body(t, carry):
            for half, p in ((1, 1), (2, 0)):
                ci = 2 * t + half
                for k in range(CHUNK):
                    b = k % NBUF
                    wait_scatter(b)
                    if k == 0:
                        wait_chunk(p)
                    if k == NBUF:
                        load_chunk(ci + 1, 1 - p, isem[1 - p])
                    gather(p, k, b)
                    b2 = (b + 2) % NBUF
                    wait_gather(b2)
                    if k >= 2:
                        scatter(p, k - 2, b2)
                    else:
                        scatter(1 - p, CHUNK - 2 + k, b2)
            return carry

        lax.fori_loop(0, 3, body, 0)

        for k in range(CHUNK):
            b = k % NBUF
            wait_scatter(b)
            if k == 0:
                wait_chunk(1)
            gather(1, k, b)
            b2 = (b + 2) % NBUF
            wait_gather(b2)
            if k >= 2:
                scatter(1, k - 2, b2)
            else:
                scatter(0, CHUNK - 2 + k, b2)
        for k in (CHUNK - 2, CHUNK - 1):
            b2 = k % NBUF
            wait_gather(b2)
            scatter(1, k, b2)
        for b in range(NBUF):
            wait_scatter(b)

        plsc.subcore_barrier()

        pltpu.sync_copy(acc_sh.at[pl.ds(row0, ROWS_PER_TILE)],
                        out_hbm.at[c, pl.ds(row0, ROWS_PER_TILE)])

    return agg_kernel(x, e5, zeros_pad)


def _tc_mlp_bn(x, partials, W1, b1, W2, b2, gamma, beta):
    def body(x_ref, p_ref, w1_ref, b1_ref, w2_ref, b2_ref, g_ref, bt_ref, o_ref):
        h = x_ref[...] + p_ref[0, :N_NODES, :] + p_ref[1, :N_NODES, :]
        h = lax.dot_general(h, w1_ref[...], (((1,), (1,)), ((), ())),
                            preferred_element_type=jnp.float32,
                            precision=lax.Precision.HIGHEST)
        h = jnp.maximum(h + b1_ref[...], 0.0)
        h = lax.dot_general(h, w2_ref[...], (((1,), (1,)), ((), ())),
                            preferred_element_type=jnp.float32,
                            precision=lax.Precision.HIGHEST)
        h = jnp.maximum(h + b2_ref[...], 0.0)
        mean = jnp.mean(h, axis=0, keepdims=True)
        var = jnp.mean(h * h, axis=0, keepdims=True) - mean * mean
        o_ref[...] = (h - mean) * lax.rsqrt(var + 1e-5) * g_ref[...] + bt_ref[...]

    return pl.pallas_call(
        body,
        out_shape=jax.ShapeDtypeStruct((N_NODES, D), jnp.float32),
    )(x, partials, W1, b1, W2, b2, gamma, beta)


def kernel(x, edge_index, W1, b1, W2, b2, gamma, beta):
    ei = edge_index.astype(jnp.int32)
    n_fill = E_PAD - N_EDGES
    pad = jnp.stack([jnp.zeros((n_fill,), jnp.int32),
                     N_NODES + (jnp.arange(n_fill, dtype=jnp.int32)
                                % (N_PAD - N_NODES))])
    e5 = (jnp.concatenate([ei, pad], axis=1)
          .reshape(2, NW, N_CHUNK, CHUNK, BATCH)
          .transpose(1, 2, 3, 0, 4))
    zeros_pad = jnp.zeros((N_PAD, D), jnp.float32)
    partials = _sc_aggregate(x, e5, zeros_pad)
    return _tc_mlp_bn(x, partials,
                      W1, b1.reshape(1, D), W2, b2.reshape(1, D),
                      gamma.reshape(1, D), beta.reshape(1, D))
